# baseline (speedup 1.0000x reference)
import functools
import math

import jax
import jax.numpy as jnp
from jax.experimental import pallas as pl

_BM = 256
_DEPTH = 3
_RATIOS = [0.75, 0.5, 0.5]


def _rup(v, m):
    return (v + m - 1) // m * m


def _pad2(a, r, c):
    return jnp.pad(a, ((0, r - a.shape[0]), (0, c - a.shape[1])))


def _degdiag_body(bt_ref, dis_ref, dfx_ref):
    i = pl.program_id(0)
    blk = bt_ref[...]
    bm, m = blk.shape
    rows = jax.lax.broadcasted_iota(jnp.int32, (bm, m), 0) + i * bm
    cols = jax.lax.broadcasted_iota(jnp.int32, (bm, m), 1)
    diag = jnp.sum(jnp.where(rows == cols, blk, 0.0), axis=1, keepdims=True)
    dfx = jnp.where(diag == 0.0, 2.0, 0.0)
    deg = jnp.sum(blk, axis=1, keepdims=True) + dfx
    dis_ref[...] = jnp.where(deg > 0.0, 1.0 / jnp.sqrt(deg), 0.0)
    dfx_ref[...] = dfx


def _degdiag(bt):
    m = bt.shape[0]
    grid = (m // _BM,)
    return pl.pallas_call(
        _degdiag_body,
        grid=grid,
        in_specs=[pl.BlockSpec((_BM, m), lambda i: (i, 0))],
        out_specs=[pl.BlockSpec((_BM, 1), lambda i: (i, 0)),
                   pl.BlockSpec((_BM, 1), lambda i: (i, 0))],
        out_shape=[jax.ShapeDtypeStruct((m, 1), jnp.float32),
                   jax.ShapeDtypeStruct((m, 1), jnp.float32)],
    )(bt)


def _zprep_body(x_ref, w_ref, dis_ref, z_ref):
    z_ref[...] = dis_ref[...] * jnp.dot(
        x_ref[...], w_ref[...], preferred_element_type=jnp.float32)


def _zprep(x, w, dis):
    m, k = x.shape
    n = w.shape[1]
    return pl.pallas_call(
        _zprep_body,
        grid=(m // _BM,),
        in_specs=[pl.BlockSpec((_BM, k), lambda i: (i, 0)),
                  pl.BlockSpec((k, n), lambda i: (0, 0)),
                  pl.BlockSpec((_BM, 1), lambda i: (i, 0))],
        out_specs=pl.BlockSpec((_BM, n), lambda i: (i, 0)),
        out_shape=jax.ShapeDtypeStruct((m, n), jnp.float32),
    )(x, w, dis)


def _gcn_body(bt_ref, z_ref, zr_ref, dis_ref, dfx_ref, b_ref, o_ref, *, act):
    acc = jnp.dot(bt_ref[...], z_ref[...], preferred_element_type=jnp.float32)
    v = dis_ref[...] * (acc + dfx_ref[...] * zr_ref[...]) + b_ref[...]
    if act == 1:
        v = jnp.maximum(v, 0.0)
    elif act == 2:
        a0 = v[:, 0:1]
        a1 = v[:, 1:2]
        mx = jnp.maximum(a0, a1)
        lse = mx + jnp.log(jnp.exp(a0 - mx) + jnp.exp(a1 - mx))
        v = v - lse
    o_ref[...] = v


def _gcn_mm(bt, z, dis, dfx, bias, act):
    m = bt.shape[0]
    n = z.shape[1]
    return pl.pallas_call(
        functools.partial(_gcn_body, act=act),
        grid=(m // _BM,),
        in_specs=[pl.BlockSpec((_BM, m), lambda i: (i, 0)),
                  pl.BlockSpec((m, n), lambda i: (0, 0)),
                  pl.BlockSpec((_BM, n), lambda i: (i, 0)),
                  pl.BlockSpec((_BM, 1), lambda i: (i, 0)),
                  pl.BlockSpec((_BM, 1), lambda i: (i, 0)),
                  pl.BlockSpec((1, n), lambda i: (0, 0))],
        out_specs=pl.BlockSpec((_BM, n), lambda i: (i, 0)),
        out_shape=jax.ShapeDtypeStruct((m, n), jnp.float32),
    )(bt, z, z, dis, dfx, bias)


def _ggh_body(g_ref, h_ref, o_ref):
    i = pl.program_id(0)
    j = pl.program_id(1)
    acc = jnp.dot(g_ref[...], h_ref[...], preferred_element_type=jnp.float32)
    bm, bn = acc.shape
    rows = jax.lax.broadcasted_iota(jnp.int32, (bm, bn), 0) + i * bm
    cols = jax.lax.broadcasted_iota(jnp.int32, (bm, bn), 1) + j * bn
    o_ref[...] = jnp.where(rows == cols, 0.0, acc)


def _ggh(g, h):
    kp, npad = g.shape
    bm = 512 if kp % 512 == 0 else _BM
    return pl.pallas_call(
        _ggh_body,
        grid=(kp // bm, kp // bm),
        in_specs=[pl.BlockSpec((bm, npad), lambda i, j: (i, 0)),
                  pl.BlockSpec((npad, bm), lambda i, j: (0, j))],
        out_specs=pl.BlockSpec((bm, bm), lambda i, j: (i, j)),
        out_shape=jax.ShapeDtypeStruct((kp, kp), jnp.float32),
    )(g, h)


def _score_body(x_ref, w_ref, s_ref):
    w = w_ref[...]
    nrm = jnp.sqrt(jnp.sum(w * w))
    s_ref[...] = jnp.tanh(
        jnp.dot(x_ref[...], w, preferred_element_type=jnp.float32) / nrm)


def _score(x, w):
    m, k = x.shape
    return pl.pallas_call(
        _score_body,
        grid=(m // _BM,),
        in_specs=[pl.BlockSpec((_BM, k), lambda i: (i, 0)),
                  pl.BlockSpec((k, 1), lambda i: (0, 0))],
        out_specs=pl.BlockSpec((_BM, 1), lambda i: (i, 0)),
        out_shape=jax.ShapeDtypeStruct((m, 1), jnp.float32),
    )(x, w)


def kernel(x, edge_index, dW0, dB0, dW1, dB1, dW2, dB2, dW3, dB3,
           pw0, pw1, pw2, uW0, uB0, uW1, uB1, uW2, uB2):
    f32 = jnp.float32
    n0 = x.shape[0]
    in_ch = x.shape[1]
    hid = dW0.shape[1]
    out_ch = uW2.shape[1]

    p0 = _rup(n0, _BM)
    cin = _rup(in_ch, 128)
    h = _rup(hid, 128)
    cout = _rup(out_ch, 128)

    ei = edge_index.astype(jnp.int32)
    bt0 = jnp.zeros((p0, p0), f32).at[ei[1], ei[0]].add(1.0)

    dWs = [_pad2(dW0, cin, h), _pad2(dW1, h, h), _pad2(dW2, h, h),
           _pad2(dW3, h, h)]
    dBs = [jnp.pad(b, (0, h - hid)).reshape(1, h) for b in (dB0, dB1, dB2, dB3)]
    pws = [jnp.pad(w, (0, h - hid)).reshape(h, 1) for w in (pw0, pw1, pw2)]
    uWs = [_pad2(uW0, h, h), _pad2(uW1, h, h), _pad2(uW2, h, cout)]
    uBs = [jnp.pad(uB0, (0, h - hid)).reshape(1, h),
           jnp.pad(uB1, (0, h - hid)).reshape(1, h),
           jnp.pad(uB2, (0, cout - out_ch)).reshape(1, cout)]

    xp = _pad2(x.astype(f32), p0, cin)

    dis0, dfx0 = _degdiag(bt0)
    z = _zprep(xp, dWs[0], dis0)
    xc = _gcn_mm(bt0, z, dis0, dfx0, dBs[0], act=1)

    xs, bts, diss, dfxs, perms = [xc], [bt0], [dis0], [dfx0], []
    nreal, npad = n0, p0
    for i in range(1, _DEPTH + 1):
        s = _score(xc, pws[i - 1])[:, 0]
        k = int(math.ceil(_RATIOS[i - 1] * nreal))
        kp = _rup(k, _BM)
        _, perm = jax.lax.top_k(s[:nreal], k)
        pad_idx = jnp.full((kp - k,), nreal, jnp.int32)
        permi = perm.astype(jnp.int32)
        permp = jnp.concatenate([permi, pad_idx])
        rngk = jnp.arange(k, dtype=jnp.int32)
        g = bts[-1][permp, :]
        g = g.at[rngk, permi].set(1.0)
        hcol = bts[-1][:, permp]
        hcol = hcol.at[permi, rngk].set(1.0)
        bt_new = _ggh(g.astype(jnp.bfloat16), hcol.astype(jnp.bfloat16))
        xg = xc[perm] * s[perm][:, None]
        xg = jnp.pad(xg, ((0, kp - k), (0, 0)))
        dis, dfx = _degdiag(bt_new)
        z = _zprep(xg, dWs[i], dis)
        xc = _gcn_mm(bt_new, z, dis, dfx, dBs[i], act=1)
        if i < _DEPTH:
            xs.append(xc)
            bts.append(bt_new)
            diss.append(dis)
            dfxs.append(dfx)
        perms.append(perm)
        nreal, npad = k, kp

    for i in range(_DEPTH):
        j = _DEPTH - 1 - i
        res, bt, perm = xs[j], bts[j], perms[j]
        kreal = perm.shape[0]
        xu = res.at[perm].add(xc[:kreal])
        act = 1 if i < _DEPTH - 1 else 2
        w = uWs[i]
        bias = uBs[i]
        z = _zprep(xu, w, diss[j])
        xc = _gcn_mm(bt, z, diss[j], dfxs[j], bias, act=act)

    return xc[:n0, :out_ch]


# speedup vs baseline: 1.3315x; 1.3315x over previous
import functools
import math

import jax
import jax.numpy as jnp
from jax.experimental import pallas as pl

_BM = 256
_DEPTH = 3
_RATIOS = [0.75, 0.5, 0.5]


def _rup(v, m):
    return (v + m - 1) // m * m


def _pad2(a, r, c):
    return jnp.pad(a, ((0, r - a.shape[0]), (0, c - a.shape[1])))


def _degsum_body(b1_ref, d_ref, dis_ref, coef_ref):
    blk = b1_ref[...].astype(jnp.float32)
    d = d_ref[...]
    dfx = jnp.where(d == 0.0, 2.0, 0.0)
    deg = jnp.sum(blk, axis=1, keepdims=True) - 1.0 + d + dfx
    dis_ref[...] = jnp.where(deg > 0.0, 1.0 / jnp.sqrt(deg), 0.0)
    coef_ref[...] = d - 1.0 + dfx


def _degsum(b1, d):
    m = b1.shape[0]
    return pl.pallas_call(
        _degsum_body,
        grid=(m // _BM,),
        in_specs=[pl.BlockSpec((_BM, m), lambda i: (i, 0)),
                  pl.BlockSpec((_BM, 1), lambda i: (i, 0))],
        out_specs=[pl.BlockSpec((_BM, 1), lambda i: (i, 0)),
                   pl.BlockSpec((_BM, 1), lambda i: (i, 0))],
        out_shape=[jax.ShapeDtypeStruct((m, 1), jnp.float32),
                   jax.ShapeDtypeStruct((m, 1), jnp.float32)],
    )(b1, d)


def _zprep_body(x_ref, w_ref, dis_ref, z_ref):
    z_ref[...] = dis_ref[...] * jnp.dot(
        x_ref[...], w_ref[...], preferred_element_type=jnp.float32)


def _zprep(x, w, dis):
    m, k = x.shape
    n = w.shape[1]
    return pl.pallas_call(
        _zprep_body,
        grid=(m // _BM,),
        in_specs=[pl.BlockSpec((_BM, k), lambda i: (i, 0)),
                  pl.BlockSpec((k, n), lambda i: (0, 0)),
                  pl.BlockSpec((_BM, 1), lambda i: (i, 0))],
        out_specs=pl.BlockSpec((_BM, n), lambda i: (i, 0)),
        out_shape=jax.ShapeDtypeStruct((m, n), jnp.float32),
    )(x, w, dis)


def _gcn_body(b1_ref, z_ref, zr_ref, dis_ref, coef_ref, b_ref, o_ref, *, act):
    bt = b1_ref[...].astype(jnp.float32)
    acc = jnp.dot(bt, z_ref[...], preferred_element_type=jnp.float32)
    v = dis_ref[...] * (acc + coef_ref[...] * zr_ref[...]) + b_ref[...]
    if act == 1:
        v = jnp.maximum(v, 0.0)
    elif act == 2:
        a0 = v[:, 0:1]
        a1 = v[:, 1:2]
        mx = jnp.maximum(a0, a1)
        lse = mx + jnp.log(jnp.exp(a0 - mx) + jnp.exp(a1 - mx))
        v = v - lse
    o_ref[...] = v


def _gcn_mm(b1, z, dis, coef, bias, act):
    m = b1.shape[0]
    n = z.shape[1]
    return pl.pallas_call(
        functools.partial(_gcn_body, act=act),
        grid=(m // _BM,),
        in_specs=[pl.BlockSpec((_BM, m), lambda i: (i, 0)),
                  pl.BlockSpec((m, n), lambda i: (0, 0)),
                  pl.BlockSpec((_BM, n), lambda i: (i, 0)),
                  pl.BlockSpec((_BM, 1), lambda i: (i, 0)),
                  pl.BlockSpec((_BM, 1), lambda i: (i, 0)),
                  pl.BlockSpec((1, n), lambda i: (0, 0))],
        out_specs=pl.BlockSpec((_BM, n), lambda i: (i, 0)),
        out_shape=jax.ShapeDtypeStruct((m, n), jnp.float32),
    )(b1, z, z, dis, coef, bias)


def _ggh_body(g_ref, h_ref, o_ref):
    i = pl.program_id(0)
    j = pl.program_id(1)
    acc = jnp.dot(g_ref[...], h_ref[...], preferred_element_type=jnp.float32)
    bm, bn = acc.shape
    rows = jax.lax.broadcasted_iota(jnp.int32, (bm, bn), 0) + i * bm
    cols = jax.lax.broadcasted_iota(jnp.int32, (bm, bn), 1) + j * bn
    o_ref[...] = jnp.where(rows == cols, 1.0, acc).astype(jnp.bfloat16)


def _ggh(g, h):
    kp, npad = g.shape
    bm = 512 if kp % 512 == 0 else _BM
    return pl.pallas_call(
        _ggh_body,
        grid=(kp // bm, kp // bm),
        in_specs=[pl.BlockSpec((bm, npad), lambda i, j: (i, 0)),
                  pl.BlockSpec((npad, bm), lambda i, j: (0, j))],
        out_specs=pl.BlockSpec((bm, bm), lambda i, j: (i, j)),
        out_shape=jax.ShapeDtypeStruct((kp, kp), jnp.bfloat16),
    )(g, h)


def _score_body(x_ref, w_ref, s_ref):
    w = w_ref[...]
    nrm = jnp.sqrt(jnp.sum(w * w))
    s_ref[...] = jnp.tanh(
        jnp.dot(x_ref[...], w, preferred_element_type=jnp.float32) / nrm)


def _score(x, w):
    m, k = x.shape
    return pl.pallas_call(
        _score_body,
        grid=(m // _BM,),
        in_specs=[pl.BlockSpec((_BM, k), lambda i: (i, 0)),
                  pl.BlockSpec((k, 1), lambda i: (0, 0))],
        out_specs=pl.BlockSpec((_BM, 1), lambda i: (i, 0)),
        out_shape=jax.ShapeDtypeStruct((m, 1), jnp.float32),
    )(x, w)


def kernel(x, edge_index, dW0, dB0, dW1, dB1, dW2, dB2, dW3, dB3,
           pw0, pw1, pw2, uW0, uB0, uW1, uB1, uW2, uB2):
    f32 = jnp.float32
    bf16 = jnp.bfloat16
    n0 = x.shape[0]
    in_ch = x.shape[1]
    hid = dW0.shape[1]
    out_ch = uW2.shape[1]

    p0 = _rup(n0, _BM)
    cin = _rup(in_ch, 128)
    h = _rup(hid, 128)
    cout = _rup(out_ch, 128)

    ei = edge_index.astype(jnp.int32)
    r, c = ei[0], ei[1]
    noself = (r != c).astype(f32)
    rows = jnp.concatenate([c, jnp.arange(n0, dtype=jnp.int32)])
    cols = jnp.concatenate([r, jnp.arange(n0, dtype=jnp.int32)])
    vals = jnp.concatenate([noself, jnp.ones((n0,), f32)])
    b1_0 = jnp.zeros((p0, p0), f32).at[rows, cols].add(vals).astype(bf16)
    d0 = jnp.zeros((p0,), f32).at[c].add(1.0 - noself).reshape(p0, 1)

    dWs = [_pad2(dW0, cin, h), _pad2(dW1, h, h), _pad2(dW2, h, h),
           _pad2(dW3, h, h)]
    dBs = [jnp.pad(b, (0, h - hid)).reshape(1, h) for b in (dB0, dB1, dB2, dB3)]
    pws = [jnp.pad(w, (0, h - hid)).reshape(h, 1) for w in (pw0, pw1, pw2)]
    uWs = [_pad2(uW0, h, h), _pad2(uW1, h, h), _pad2(uW2, h, cout)]
    uBs = [jnp.pad(uB0, (0, h - hid)).reshape(1, h),
           jnp.pad(uB1, (0, h - hid)).reshape(1, h),
           jnp.pad(uB2, (0, cout - out_ch)).reshape(1, cout)]

    xp = _pad2(x.astype(f32), p0, cin)

    dis0, coef0 = _degsum(b1_0, d0)
    z = _zprep(xp, dWs[0], dis0)
    xc = _gcn_mm(b1_0, z, dis0, coef0, dBs[0], act=1)

    xs, bts, diss, coefs, perms = [xc], [b1_0], [dis0], [coef0], []
    nreal = n0
    for i in range(1, _DEPTH + 1):
        s = _score(xc, pws[i - 1])[:, 0]
        k = int(math.ceil(_RATIOS[i - 1] * nreal))
        kp = _rup(k, _BM)
        _, perm = jax.lax.top_k(s[:nreal], k)
        pad_idx = jnp.full((kp - k,), nreal, jnp.int32)
        permp = jnp.concatenate([perm.astype(jnp.int32), pad_idx])
        g = bts[-1][permp, :]
        hcol = bts[-1][:, permp]
        bt_new = _ggh(g, hcol)
        xg = xc[perm] * s[perm][:, None]
        xg = jnp.pad(xg, ((0, kp - k), (0, 0)))
        zd = jnp.zeros((kp, 1), f32)
        dis, coef = _degsum(bt_new, zd)
        z = _zprep(xg, dWs[i], dis)
        xc = _gcn_mm(bt_new, z, dis, coef, dBs[i], act=1)
        if i < _DEPTH:
            xs.append(xc)
            bts.append(bt_new)
            diss.append(dis)
            coefs.append(coef)
        perms.append(perm)
        nreal = k

    for i in range(_DEPTH):
        j = _DEPTH - 1 - i
        res, bt, perm = xs[j], bts[j], perms[j]
        kreal = perm.shape[0]
        xu = res.at[perm].add(xc[:kreal])
        act = 1 if i < _DEPTH - 1 else 2
        z = _zprep(xu, uWs[i], diss[j])
        xc = _gcn_mm(bt, z, diss[j], coefs[j], uBs[i], act=act)

    return xc[:n0, :out_ch]


# speedup vs baseline: 1.3432x; 1.0088x over previous
import functools
import math

import jax
import jax.numpy as jnp
from jax.experimental import pallas as pl

_BM = 256
_DEPTH = 3
_RATIOS = [0.75, 0.5, 0.5]


def _rup(v, m):
    return (v + m - 1) // m * m


def _pad2(a, r, c):
    return jnp.pad(a, ((0, r - a.shape[0]), (0, c - a.shape[1])))


def _degsum_body(b1_ref, d_ref, dis_ref, coef_ref):
    blk = b1_ref[...].astype(jnp.float32)
    d = d_ref[...]
    dfx = jnp.where(d == 0.0, 2.0, 0.0)
    deg = jnp.sum(blk, axis=1, keepdims=True) - 1.0 + d + dfx
    dis_ref[...] = jnp.where(deg > 0.0, 1.0 / jnp.sqrt(deg), 0.0)
    coef_ref[...] = d - 1.0 + dfx


def _degsum(b1, d):
    m = b1.shape[0]
    return pl.pallas_call(
        _degsum_body,
        grid=(m // _BM,),
        in_specs=[pl.BlockSpec((_BM, m), lambda i: (i, 0)),
                  pl.BlockSpec((_BM, 1), lambda i: (i, 0))],
        out_specs=[pl.BlockSpec((_BM, 1), lambda i: (i, 0)),
                   pl.BlockSpec((_BM, 1), lambda i: (i, 0))],
        out_shape=[jax.ShapeDtypeStruct((m, 1), jnp.float32),
                   jax.ShapeDtypeStruct((m, 1), jnp.float32)],
    )(b1, d)


def _zprep_body(x_ref, w_ref, dis_ref, z_ref):
    z_ref[...] = dis_ref[...] * jnp.dot(
        x_ref[...], w_ref[...], preferred_element_type=jnp.float32)


def _zprep(x, w, dis):
    m, k = x.shape
    n = w.shape[1]
    return pl.pallas_call(
        _zprep_body,
        grid=(m // _BM,),
        in_specs=[pl.BlockSpec((_BM, k), lambda i: (i, 0)),
                  pl.BlockSpec((k, n), lambda i: (0, 0)),
                  pl.BlockSpec((_BM, 1), lambda i: (i, 0))],
        out_specs=pl.BlockSpec((_BM, n), lambda i: (i, 0)),
        out_shape=jax.ShapeDtypeStruct((m, n), jnp.float32),
    )(x, w, dis)


def _gcn_body(b1_ref, z_ref, zr_ref, dis_ref, coef_ref, b_ref, o_ref, *, act):
    bt = b1_ref[...].astype(jnp.float32)
    acc = jnp.dot(bt, z_ref[...], preferred_element_type=jnp.float32)
    v = dis_ref[...] * (acc + coef_ref[...] * zr_ref[...]) + b_ref[...]
    if act == 1:
        v = jnp.maximum(v, 0.0)
    elif act == 2:
        a0 = v[:, 0:1]
        a1 = v[:, 1:2]
        mx = jnp.maximum(a0, a1)
        lse = mx + jnp.log(jnp.exp(a0 - mx) + jnp.exp(a1 - mx))
        v = v - lse
    o_ref[...] = v


def _gcn_mm(b1, z, dis, coef, bias, act):
    m = b1.shape[0]
    n = z.shape[1]
    bm = 512 if m % 512 == 0 else _BM
    return pl.pallas_call(
        functools.partial(_gcn_body, act=act),
        grid=(m // bm,),
        in_specs=[pl.BlockSpec((bm, m), lambda i: (i, 0)),
                  pl.BlockSpec((m, n), lambda i: (0, 0)),
                  pl.BlockSpec((bm, n), lambda i: (i, 0)),
                  pl.BlockSpec((bm, 1), lambda i: (i, 0)),
                  pl.BlockSpec((bm, 1), lambda i: (i, 0)),
                  pl.BlockSpec((1, n), lambda i: (0, 0))],
        out_specs=pl.BlockSpec((bm, n), lambda i: (i, 0)),
        out_shape=jax.ShapeDtypeStruct((m, n), jnp.float32),
    )(b1, z, z, dis, coef, bias)


def _ggh_body(g_ref, h_ref, o_ref):
    i = pl.program_id(0)
    j = pl.program_id(1)
    acc = jnp.dot(g_ref[...], h_ref[...], preferred_element_type=jnp.float32)
    bm, bn = acc.shape
    rows = jax.lax.broadcasted_iota(jnp.int32, (bm, bn), 0) + i * bm
    cols = jax.lax.broadcasted_iota(jnp.int32, (bm, bn), 1) + j * bn
    o_ref[...] = jnp.where(rows == cols, 1.0, acc).astype(jnp.bfloat16)


def _ggh(g, h):
    kp, npad = g.shape
    bm = 1024 if kp % 1024 == 0 else (512 if kp % 512 == 0 else _BM)
    return pl.pallas_call(
        _ggh_body,
        grid=(kp // bm, kp // bm),
        in_specs=[pl.BlockSpec((bm, npad), lambda i, j: (i, 0)),
                  pl.BlockSpec((npad, bm), lambda i, j: (0, j))],
        out_specs=pl.BlockSpec((bm, bm), lambda i, j: (i, j)),
        out_shape=jax.ShapeDtypeStruct((kp, kp), jnp.bfloat16),
    )(g, h)


def _score_body(x_ref, w_ref, s_ref):
    w = w_ref[...]
    nrm = jnp.sqrt(jnp.sum(w * w))
    s_ref[...] = jnp.tanh(
        jnp.dot(x_ref[...], w, preferred_element_type=jnp.float32) / nrm)


def _score(x, w):
    m, k = x.shape
    return pl.pallas_call(
        _score_body,
        grid=(m // _BM,),
        in_specs=[pl.BlockSpec((_BM, k), lambda i: (i, 0)),
                  pl.BlockSpec((k, 1), lambda i: (0, 0))],
        out_specs=pl.BlockSpec((_BM, 1), lambda i: (i, 0)),
        out_shape=jax.ShapeDtypeStruct((m, 1), jnp.float32),
    )(x, w)


def kernel(x, edge_index, dW0, dB0, dW1, dB1, dW2, dB2, dW3, dB3,
           pw0, pw1, pw2, uW0, uB0, uW1, uB1, uW2, uB2):
    f32 = jnp.float32
    bf16 = jnp.bfloat16
    n0 = x.shape[0]
    in_ch = x.shape[1]
    hid = dW0.shape[1]
    out_ch = uW2.shape[1]

    p0 = _rup(n0, _BM)
    cin = _rup(in_ch, 128)
    h = _rup(hid, 128)
    cout = _rup(out_ch, 128)

    ei = edge_index.astype(jnp.int32)
    r, c = ei[0], ei[1]
    noself = (r != c).astype(f32)
    rows = jnp.concatenate([c, jnp.arange(n0, dtype=jnp.int32)])
    cols = jnp.concatenate([r, jnp.arange(n0, dtype=jnp.int32)])
    vals = jnp.concatenate([noself, jnp.ones((n0,), f32)])
    b1_0 = jnp.zeros((p0, p0), f32).at[rows, cols].add(vals).astype(bf16)
    d0 = jnp.zeros((p0,), f32).at[c].add(1.0 - noself).reshape(p0, 1)

    dWs = [_pad2(dW0, cin, h), _pad2(dW1, h, h), _pad2(dW2, h, h),
           _pad2(dW3, h, h)]
    dBs = [jnp.pad(b, (0, h - hid)).reshape(1, h) for b in (dB0, dB1, dB2, dB3)]
    pws = [jnp.pad(w, (0, h - hid)).reshape(h, 1) for w in (pw0, pw1, pw2)]
    uWs = [_pad2(uW0, h, h), _pad2(uW1, h, h), _pad2(uW2, h, cout)]
    uBs = [jnp.pad(uB0, (0, h - hid)).reshape(1, h),
           jnp.pad(uB1, (0, h - hid)).reshape(1, h),
           jnp.pad(uB2, (0, cout - out_ch)).reshape(1, cout)]

    xp = _pad2(x.astype(f32), p0, cin)

    dis0, coef0 = _degsum(b1_0, d0)
    z = _zprep(xp, dWs[0], dis0)
    xc = _gcn_mm(b1_0, z, dis0, coef0, dBs[0], act=1)

    xs, bts, diss, coefs, perms = [xc], [b1_0], [dis0], [coef0], []
    nreal = n0
    for i in range(1, _DEPTH + 1):
        s = _score(xc, pws[i - 1])[:, 0]
        k = int(math.ceil(_RATIOS[i - 1] * nreal))
        kp = _rup(k, _BM)
        _, perm = jax.lax.top_k(s[:nreal], k)
        pad_idx = jnp.full((kp - k,), nreal, jnp.int32)
        permp = jnp.concatenate([perm.astype(jnp.int32), pad_idx])
        g = bts[-1][permp, :]
        hcol = bts[-1][:, permp]
        bt_new = _ggh(g, hcol)
        xg = xc[perm] * s[perm][:, None]
        xg = jnp.pad(xg, ((0, kp - k), (0, 0)))
        zd = jnp.zeros((kp, 1), f32)
        dis, coef = _degsum(bt_new, zd)
        z = _zprep(xg, dWs[i], dis)
        xc = _gcn_mm(bt_new, z, dis, coef, dBs[i], act=1)
        if i < _DEPTH:
            xs.append(xc)
            bts.append(bt_new)
            diss.append(dis)
            coefs.append(coef)
        perms.append(perm)
        nreal = k

    for i in range(_DEPTH):
        j = _DEPTH - 1 - i
        res, bt, perm = xs[j], bts[j], perms[j]
        kreal = perm.shape[0]
        xu = res.at[perm].add(xc[:kreal])
        act = 1 if i < _DEPTH - 1 else 2
        z = _zprep(xu, uWs[i], diss[j])
        xc = _gcn_mm(bt, z, diss[j], coefs[j], uBs[i], act=act)

    return xc[:n0, :out_ch]


# speedup vs baseline: 1.4426x; 1.0740x over previous
import functools
import math

import jax
import jax.numpy as jnp
from jax.experimental import pallas as pl

_BM = 256
_DEPTH = 3
_RATIOS = [0.75, 0.5, 0.5]


def _rup(v, m):
    return (v + m - 1) // m * m


def _pad2(a, r, c):
    return jnp.pad(a, ((0, r - a.shape[0]), (0, c - a.shape[1])))


def _degsum_body(b1_ref, d_ref, dis_ref, coef_ref):
    blk = b1_ref[...].astype(jnp.float32)
    d = d_ref[...]
    dfx = jnp.where(d == 0.0, 2.0, 0.0)
    deg = jnp.sum(blk, axis=1, keepdims=True) - 1.0 + d + dfx
    dis_ref[...] = jnp.where(deg > 0.0, 1.0 / jnp.sqrt(deg), 0.0)
    coef_ref[...] = d - 1.0 + dfx


def _degsum(b1, d):
    m = b1.shape[0]
    return pl.pallas_call(
        _degsum_body,
        grid=(m // _BM,),
        in_specs=[pl.BlockSpec((_BM, m), lambda i: (i, 0)),
                  pl.BlockSpec((_BM, 1), lambda i: (i, 0))],
        out_specs=[pl.BlockSpec((_BM, 1), lambda i: (i, 0)),
                   pl.BlockSpec((_BM, 1), lambda i: (i, 0))],
        out_shape=[jax.ShapeDtypeStruct((m, 1), jnp.float32),
                   jax.ShapeDtypeStruct((m, 1), jnp.float32)],
    )(b1, d)


def _zprep_body(x_ref, w_ref, dis_ref, z_ref):
    z_ref[...] = dis_ref[...] * jnp.dot(
        x_ref[...], w_ref[...], preferred_element_type=jnp.float32)


def _zprep(x, w, dis):
    m, k = x.shape
    n = w.shape[1]
    return pl.pallas_call(
        _zprep_body,
        grid=(m // _BM,),
        in_specs=[pl.BlockSpec((_BM, k), lambda i: (i, 0)),
                  pl.BlockSpec((k, n), lambda i: (0, 0)),
                  pl.BlockSpec((_BM, 1), lambda i: (i, 0))],
        out_specs=pl.BlockSpec((_BM, n), lambda i: (i, 0)),
        out_shape=jax.ShapeDtypeStruct((m, n), jnp.float32),
    )(x, w, dis)


def _gcn_body(b1_ref, z_ref, zr_ref, dis_ref, coef_ref, b_ref, o_ref, *, act):
    bt = b1_ref[...].astype(jnp.float32)
    acc = jnp.dot(bt, z_ref[...], preferred_element_type=jnp.float32)
    v = dis_ref[...] * (acc + coef_ref[...] * zr_ref[...]) + b_ref[...]
    if act == 1:
        v = jnp.maximum(v, 0.0)
    elif act == 2:
        a0 = v[:, 0:1]
        a1 = v[:, 1:2]
        mx = jnp.maximum(a0, a1)
        lse = mx + jnp.log(jnp.exp(a0 - mx) + jnp.exp(a1 - mx))
        v = v - lse
    o_ref[...] = v


def _gcn_mm(b1, z, dis, coef, bias, act):
    m = b1.shape[0]
    n = z.shape[1]
    bm = 512 if m % 512 == 0 else _BM
    return pl.pallas_call(
        functools.partial(_gcn_body, act=act),
        grid=(m // bm,),
        in_specs=[pl.BlockSpec((bm, m), lambda i: (i, 0)),
                  pl.BlockSpec((m, n), lambda i: (0, 0)),
                  pl.BlockSpec((bm, n), lambda i: (i, 0)),
                  pl.BlockSpec((bm, 1), lambda i: (i, 0)),
                  pl.BlockSpec((bm, 1), lambda i: (i, 0)),
                  pl.BlockSpec((1, n), lambda i: (0, 0))],
        out_specs=pl.BlockSpec((bm, n), lambda i: (i, 0)),
        out_shape=jax.ShapeDtypeStruct((m, n), jnp.float32),
    )(b1, z, z, dis, coef, bias)


def _ggh_body(g_ref, h_ref, o_ref):
    i = pl.program_id(0)
    j = pl.program_id(1)
    acc = jnp.dot(g_ref[...], h_ref[...], preferred_element_type=jnp.float32)
    bm, bn = acc.shape
    rows = jax.lax.broadcasted_iota(jnp.int32, (bm, bn), 0) + i * bm
    cols = jax.lax.broadcasted_iota(jnp.int32, (bm, bn), 1) + j * bn
    o_ref[...] = jnp.where(rows == cols, 1.0, acc).astype(jnp.bfloat16)


def _ggh(g, h):
    kp, npad = g.shape
    bm = 1024 if kp % 1024 == 0 else (512 if kp % 512 == 0 else _BM)
    return pl.pallas_call(
        _ggh_body,
        grid=(kp // bm, kp // bm),
        in_specs=[pl.BlockSpec((bm, npad), lambda i, j: (i, 0)),
                  pl.BlockSpec((npad, bm), lambda i, j: (0, j))],
        out_specs=pl.BlockSpec((bm, bm), lambda i, j: (i, j)),
        out_shape=jax.ShapeDtypeStruct((kp, kp), jnp.bfloat16),
    )(g, h)


def _score_body(x_ref, w_ref, s_ref):
    w = w_ref[...]
    nrm = jnp.sqrt(jnp.sum(w * w))
    s_ref[...] = jnp.tanh(
        jnp.dot(x_ref[...], w, preferred_element_type=jnp.float32) / nrm)


def _score(x, w):
    m, k = x.shape
    return pl.pallas_call(
        _score_body,
        grid=(m // _BM,),
        in_specs=[pl.BlockSpec((_BM, k), lambda i: (i, 0)),
                  pl.BlockSpec((k, 1), lambda i: (0, 0))],
        out_specs=pl.BlockSpec((_BM, 1), lambda i: (i, 0)),
        out_shape=jax.ShapeDtypeStruct((m, 1), jnp.float32),
    )(x, w)


def kernel(x, edge_index, dW0, dB0, dW1, dB1, dW2, dB2, dW3, dB3,
           pw0, pw1, pw2, uW0, uB0, uW1, uB1, uW2, uB2):
    f32 = jnp.float32
    bf16 = jnp.bfloat16
    n0 = x.shape[0]
    in_ch = x.shape[1]
    hid = dW0.shape[1]
    out_ch = uW2.shape[1]

    p0 = _rup(n0, _BM)
    cin = _rup(in_ch, 128)
    h = _rup(hid, 128)
    cout = _rup(out_ch, 128)

    ei = edge_index.astype(jnp.int32)
    r, c = ei[0], ei[1]
    noself = (r != c).astype(f32)
    rows = jnp.concatenate([c, jnp.arange(n0, dtype=jnp.int32)])
    cols = jnp.concatenate([r, jnp.arange(n0, dtype=jnp.int32)])
    vals = jnp.concatenate([noself, jnp.ones((n0,), f32)])
    b1_0 = jnp.zeros((p0, p0), f32).at[rows, cols].add(vals).astype(bf16)
    d0 = jnp.zeros((p0,), f32).at[c].add(1.0 - noself).reshape(p0, 1)

    dWs = [_pad2(dW0, cin, h), _pad2(dW1, h, h), _pad2(dW2, h, h),
           _pad2(dW3, h, h)]
    dBs = [jnp.pad(b, (0, h - hid)).reshape(1, h) for b in (dB0, dB1, dB2, dB3)]
    pws = [jnp.pad(w, (0, h - hid)).reshape(h, 1) for w in (pw0, pw1, pw2)]
    uWs = [_pad2(uW0, h, h), _pad2(uW1, h, h), _pad2(uW2, h, cout)]
    uBs = [jnp.pad(uB0, (0, h - hid)).reshape(1, h),
           jnp.pad(uB1, (0, h - hid)).reshape(1, h),
           jnp.pad(uB2, (0, cout - out_ch)).reshape(1, cout)]

    xp = _pad2(x.astype(f32), p0, cin)

    dis0, coef0 = _degsum(b1_0, d0)
    z = _zprep(xp, dWs[0], dis0)
    xc = _gcn_mm(b1_0, z, dis0, coef0, dBs[0], act=1)

    xs, bts, diss, coefs, perms = [xc], [b1_0], [dis0], [coef0], []
    nreal = n0
    for i in range(1, _DEPTH + 1):
        s = _score(xc, pws[i - 1])[:, 0]
        k = int(math.ceil(_RATIOS[i - 1] * nreal))
        kp = _rup(k, _BM)
        _, perm = jax.lax.top_k(s[:nreal], k)
        pad_idx = jnp.full((kp - k,), nreal, jnp.int32)
        permp = jnp.concatenate([perm.astype(jnp.int32), pad_idx])
        g = bts[-1][permp, :]
        hcol = bts[-1][:, permp]
        bt_new = _ggh(g, hcol)
        xg = xc[perm] * s[perm][:, None]
        xg = jnp.pad(xg, ((0, kp - k), (0, 0)))
        zd = jnp.zeros((kp, 1), f32)
        dis, coef = _degsum(bt_new, zd)
        z = _zprep(xg, dWs[i], dis)
        xc = _gcn_mm(bt_new, z, dis, coef, dBs[i], act=1)
        if i < _DEPTH:
            xs.append(xc)
            bts.append(bt_new)
            diss.append(dis)
            coefs.append(coef)
        perms.append(perm)
        nreal = k

    for i in range(_DEPTH):
        j = _DEPTH - 1 - i
        res, bt, perm = xs[j], bts[j], perms[j]
        kreal = perm.shape[0]
        mj = res.shape[0]
        inv = jnp.full((mj,), -1, jnp.int32).at[perm].set(
            jnp.arange(kreal, dtype=jnp.int32))
        up = jnp.where((inv >= 0)[:, None], xc[jnp.maximum(inv, 0)], 0.0)
        xu = res + up
        act = 1 if i < _DEPTH - 1 else 2
        z = _zprep(xu, uWs[i], diss[j])
        xc = _gcn_mm(bt, z, diss[j], coefs[j], uBs[i], act=act)

    return xc[:n0, :out_ch]
